# baseline (speedup 1.0000x reference)
import functools

import jax
import jax.numpy as jnp
from jax import lax
from jax.experimental import pallas as pl
from jax.experimental.pallas import tpu as pltpu
from jax.experimental.pallas import tpu_sc as plsc

N_V = 1000000
N_D = 64
N_DP = 128
N_DS = 129
BATCH = 16384
HIST = 50
L = 16

_info = plsc.get_sparse_core_info()
NC, NS = _info.num_cores, _info.num_subcores
NW = NC * NS
CB = 128
NCB = BATCH // CB
N_CHUNKS = HIST * NCB
CHUNKS_PER_W = N_CHUNKS // NW
PAIRS_PER_W = CHUNKS_PER_W // 2


def _make_gather():
    mesh = plsc.VectorSubcoreMesh(core_axis_name="c", subcore_axis_name="s")

    @functools.partial(
        pl.kernel,
        mesh=mesh,
        out_type=jax.ShapeDtypeStruct((HIST, N_D, BATCH), jnp.float32),
        compiler_params=pltpu.CompilerParams(needs_layout_passes=False),
        scratch_types=[
            pltpu.VMEM((CB,), jnp.int32),
            pltpu.VMEM((CB,), jnp.int32),
            pltpu.VMEM((CB, N_DS), jnp.float32),
            pltpu.VMEM((CB, N_DS), jnp.float32),
            pltpu.VMEM((N_D, CB), jnp.float32),
            pltpu.VMEM((N_D, CB), jnp.float32),
            pltpu.SemaphoreType.DMA,
            pltpu.SemaphoreType.DMA,
            pltpu.SemaphoreType.DMA,
            pltpu.SemaphoreType.DMA,
        ],
    )
    def gather_kernel(
        idx_hbm, table_hbm, out_hbm,
        idx0, idx1, rows0, rows1, outb0, outb1,
        sg0, sg1, so0, so1,
    ):
        wid = lax.axis_index("s") * NC + lax.axis_index("c")
        base = wid * CHUNKS_PER_W
        row_iotas = [lax.iota(jnp.int32, L) + jg * L for jg in range(CB // L)]

        def chunk_hb(c):
            h = c // NCB
            b0 = (c % NCB) * CB
            return h, b0

        def stage(c, idx_v, rows_v, sem):
            h, b0 = chunk_hb(c)
            pltpu.sync_copy(idx_hbm.at[h, pl.ds(b0, CB)], idx_v)
            pltpu.async_copy(
                table_hbm.at[idx_v], rows_v.at[slice(None), pl.ds(0, N_DP)], sem
            )

        def transpose(rows_v, outb_v):
            def trans_d(d, carry):
                col = lax.iota(jnp.int32, L) * 0 + d
                for jg in range(CB // L):
                    v = plsc.load_gather(rows_v, [row_iotas[jg], col])
                    outb_v[d, pl.ds(jg * L, L)] = v
                return carry

            lax.fori_loop(0, N_D, trans_d, 0)

        def write(c, outb_v, sem):
            h, b0 = chunk_hb(c)
            pltpu.async_copy(outb_v, out_hbm.at[h, slice(None), pl.ds(b0, CB)], sem)

        def wait_gather(c, idx_v, rows_v, sem):
            pltpu.make_async_copy(
                table_hbm.at[idx_v], rows_v.at[slice(None), pl.ds(0, N_DP)], sem
            ).wait()

        def wait_write(c, outb_v, sem):
            h, b0 = chunk_hb(c)
            pltpu.make_async_copy(
                outb_v, out_hbm.at[h, slice(None), pl.ds(b0, CB)], sem
            ).wait()

        stage(base, idx0, rows0, sg0)

        def step(g, carry):
            c0 = base + 2 * g
            c1 = c0 + 1
            c2 = c0 + 2
            stage(c1, idx1, rows1, sg1)
            wait_gather(c0, idx0, rows0, sg0)

            @pl.when(g > 0)
            def _():
                wait_write(c0 - 2, outb0, so0)

            transpose(rows0, outb0)
            write(c0, outb0, so0)

            @pl.when(g < PAIRS_PER_W - 1)
            def _():
                stage(c2, idx0, rows0, sg0)

            wait_gather(c1, idx1, rows1, sg1)

            @pl.when(g > 0)
            def _():
                wait_write(c1 - 2, outb1, so1)

            transpose(rows1, outb1)
            write(c1, outb1, so1)
            return carry

        lax.fori_loop(0, PAIRS_PER_W, step, 0)
        last0 = base + CHUNKS_PER_W - 2
        wait_write(last0, outb0, so0)
        wait_write(last0 + 1, outb1, so1)

    return gather_kernel


_gather = _make_gather()


def kernel(input, weight):
    wp = jnp.pad(weight, ((0, 0), (0, N_DP - N_D)))
    out_t = _gather(input.T, wp)
    return jnp.transpose(out_t, (2, 0, 1))


# speedup vs baseline: 1.1379x; 1.1379x over previous
import functools

import jax
import jax.numpy as jnp
from jax import lax
from jax.experimental import pallas as pl
from jax.experimental.pallas import tpu as pltpu
from jax.experimental.pallas import tpu_sc as plsc

N_V = 1000000
N_D = 64
N_DP = 128
N_DS = 129
BATCH = 16384
HIST = 50
L = 16

_info = plsc.get_sparse_core_info()
NC, NS = _info.num_cores, _info.num_subcores
NW = NC * NS
CB = 128
NCB = BATCH // CB
N_CHUNKS = HIST * NCB
CHUNKS_PER_W = N_CHUNKS // NW
PAIRS_PER_W = CHUNKS_PER_W // 2


def _make_gather():
    mesh = plsc.VectorSubcoreMesh(core_axis_name="c", subcore_axis_name="s")

    @functools.partial(
        pl.kernel,
        mesh=mesh,
        out_type=jax.ShapeDtypeStruct((HIST, N_D, BATCH), jnp.float32),
        compiler_params=pltpu.CompilerParams(needs_layout_passes=False),
        scratch_types=[
            pltpu.VMEM((CB,), jnp.int32),
            pltpu.VMEM((CB,), jnp.int32),
            pltpu.VMEM((CB, N_DS), jnp.float32),
            pltpu.VMEM((CB, N_DS), jnp.float32),
            pltpu.VMEM((N_D, CB + 1), jnp.float32),
            pltpu.VMEM((N_D, CB + 1), jnp.float32),
            pltpu.SemaphoreType.DMA,
            pltpu.SemaphoreType.DMA,
            pltpu.SemaphoreType.DMA,
            pltpu.SemaphoreType.DMA,
        ],
    )
    def gather_kernel(
        idx_hbm, table_hbm, out_hbm,
        idx0, idx1, rows0, rows1, outb0, outb1,
        sg0, sg1, so0, so1,
    ):
        wid = lax.axis_index("s") * NC + lax.axis_index("c")
        base = wid * CHUNKS_PER_W
        row_iotas = [lax.iota(jnp.int32, L) + jg * L for jg in range(CB // L)]

        def chunk_hb(c):
            h = c // NCB
            b0 = (c % NCB) * CB
            return h, b0

        def stage(c, idx_v, rows_v, sem):
            h, b0 = chunk_hb(c)
            pltpu.sync_copy(idx_hbm.at[h, pl.ds(b0, CB)], idx_v)
            pltpu.async_copy(
                table_hbm.at[idx_v], rows_v.at[slice(None), pl.ds(0, N_DP)], sem
            )

        def transpose(rows_v, outb_v):
            d_iotas = [lax.iota(jnp.int32, L) + k * L for k in range(N_D // L)]

            def trans_j(j, carry):
                colj = lax.iota(jnp.int32, L) * 0 + j
                for k in range(N_D // L):
                    v = rows_v[j, pl.ds(k * L, L)]
                    plsc.store_scatter(outb_v, [d_iotas[k], colj], v)
                return carry

            lax.fori_loop(0, CB, trans_j, 0)

        def write(c, outb_v, sem):
            h, b0 = chunk_hb(c)
            pltpu.async_copy(
                outb_v.at[slice(None), pl.ds(0, CB)],
                out_hbm.at[h, slice(None), pl.ds(b0, CB)],
                sem,
            )

        def wait_gather(c, idx_v, rows_v, sem):
            pltpu.make_async_copy(
                table_hbm.at[idx_v], rows_v.at[slice(None), pl.ds(0, N_DP)], sem
            ).wait()

        def wait_write(c, outb_v, sem):
            h, b0 = chunk_hb(c)
            pltpu.make_async_copy(
                outb_v.at[slice(None), pl.ds(0, CB)],
                out_hbm.at[h, slice(None), pl.ds(b0, CB)],
                sem,
            ).wait()

        stage(base, idx0, rows0, sg0)

        def step(g, carry):
            c0 = base + 2 * g
            c1 = c0 + 1
            c2 = c0 + 2
            stage(c1, idx1, rows1, sg1)
            wait_gather(c0, idx0, rows0, sg0)

            @pl.when(g > 0)
            def _():
                wait_write(c0 - 2, outb0, so0)

            transpose(rows0, outb0)
            write(c0, outb0, so0)

            @pl.when(g < PAIRS_PER_W - 1)
            def _():
                stage(c2, idx0, rows0, sg0)

            wait_gather(c1, idx1, rows1, sg1)

            @pl.when(g > 0)
            def _():
                wait_write(c1 - 2, outb1, so1)

            transpose(rows1, outb1)
            write(c1, outb1, so1)
            return carry

        lax.fori_loop(0, PAIRS_PER_W, step, 0)
        last0 = base + CHUNKS_PER_W - 2
        wait_write(last0, outb0, so0)
        wait_write(last0 + 1, outb1, so1)

    return gather_kernel


_gather = _make_gather()


def kernel(input, weight):
    wp = jnp.pad(weight, ((0, 0), (0, N_DP - N_D)))
    out_t = _gather(input.T, wp)
    return jnp.transpose(out_t, (2, 0, 1))


# speedup vs baseline: 2.0642x; 1.8141x over previous
import functools

import jax
import jax.numpy as jnp
from jax import lax
from jax.experimental import pallas as pl
from jax.experimental.pallas import tpu as pltpu
from jax.experimental.pallas import tpu_sc as plsc

N_V = 1000000
N_D = 64
N_DP = 128
N_DS = 129
BATCH = 16384
HIST = 50
L = 16

_info = plsc.get_sparse_core_info()
NC, NS = _info.num_cores, _info.num_subcores
NW = NC * NS
CB = 128
NCB = BATCH // CB
N_CHUNKS = HIST * NCB
CHUNKS_PER_W = N_CHUNKS // NW
PAIRS_PER_W = CHUNKS_PER_W // 2


def _make_gather():
    mesh = plsc.VectorSubcoreMesh(core_axis_name="c", subcore_axis_name="s")

    @functools.partial(
        pl.kernel,
        mesh=mesh,
        out_type=jax.ShapeDtypeStruct((HIST, N_D, BATCH), jnp.float32),
        compiler_params=pltpu.CompilerParams(needs_layout_passes=False),
        scratch_types=[
            pltpu.VMEM((CB,), jnp.int32),
            pltpu.VMEM((CB,), jnp.int32),
            pltpu.VMEM((CB, N_DS), jnp.float32),
            pltpu.VMEM((CB, N_DS), jnp.float32),
            pltpu.VMEM((N_D, CB + 1), jnp.float32),
            pltpu.VMEM((N_D, CB + 1), jnp.float32),
            pltpu.SemaphoreType.DMA,
            pltpu.SemaphoreType.DMA,
            pltpu.SemaphoreType.DMA,
            pltpu.SemaphoreType.DMA,
        ],
    )
    def gather_kernel(
        idx_hbm, table_hbm, out_hbm,
        idx0, idx1, rows0, rows1, outb0, outb1,
        sg0, sg1, so0, so1,
    ):
        wid = lax.axis_index("s") * NC + lax.axis_index("c")
        base = wid * CHUNKS_PER_W
        row_iotas = [lax.iota(jnp.int32, L) + jg * L for jg in range(CB // L)]

        def chunk_hb(c):
            h = c // NCB
            b0 = (c % NCB) * CB
            return h, b0

        def stage(c, idx_v, rows_v, sem):
            h, b0 = chunk_hb(c)
            pltpu.sync_copy(idx_hbm.at[h, pl.ds(b0, CB)], idx_v)
            pltpu.async_copy(
                table_hbm.at[idx_v], rows_v.at[slice(None), pl.ds(0, N_DP)], sem
            )

        lane = lax.iota(jnp.int32, L)
        perms = {s: jnp.bitwise_xor(lane, s) for s in (1, 2, 4, 8)}
        masks = {s: jnp.bitwise_and(lane, s) == 0 for s in (1, 2, 4, 8)}

        def transpose(rows_v, outb_v):
            def trans_jb(jb, carry):
                j0 = jb * L
                for dg in range(N_D // L):
                    d0 = dg * L
                    regs = [rows_v[j0 + jj, pl.ds(d0, L)] for jj in range(L)]
                    for s in (1, 2, 4, 8):
                        m, perm = masks[s], perms[s]
                        for p in range(L):
                            if p & s:
                                continue
                            q = p | s
                            a, b = regs[p], regs[q]
                            ta = jnp.take_along_axis(a, perm, axis=0)
                            tb = jnp.take_along_axis(b, perm, axis=0)
                            regs[p] = jnp.where(m, a, tb)
                            regs[q] = jnp.where(m, ta, b)
                    for dd in range(L):
                        outb_v[d0 + dd, pl.ds(j0, L)] = regs[dd]
                return carry

            lax.fori_loop(0, CB // L, trans_jb, 0)

        def write(c, outb_v, sem):
            h, b0 = chunk_hb(c)
            pltpu.async_copy(
                outb_v.at[slice(None), pl.ds(0, CB)],
                out_hbm.at[h, slice(None), pl.ds(b0, CB)],
                sem,
            )

        def wait_gather(c, idx_v, rows_v, sem):
            pltpu.make_async_copy(
                table_hbm.at[idx_v], rows_v.at[slice(None), pl.ds(0, N_DP)], sem
            ).wait()

        def wait_write(c, outb_v, sem):
            h, b0 = chunk_hb(c)
            pltpu.make_async_copy(
                outb_v.at[slice(None), pl.ds(0, CB)],
                out_hbm.at[h, slice(None), pl.ds(b0, CB)],
                sem,
            ).wait()

        stage(base, idx0, rows0, sg0)

        def step(g, carry):
            c0 = base + 2 * g
            c1 = c0 + 1
            c2 = c0 + 2
            stage(c1, idx1, rows1, sg1)
            wait_gather(c0, idx0, rows0, sg0)

            @pl.when(g > 0)
            def _():
                wait_write(c0 - 2, outb0, so0)

            transpose(rows0, outb0)
            write(c0, outb0, so0)

            @pl.when(g < PAIRS_PER_W - 1)
            def _():
                stage(c2, idx0, rows0, sg0)

            wait_gather(c1, idx1, rows1, sg1)

            @pl.when(g > 0)
            def _():
                wait_write(c1 - 2, outb1, so1)

            transpose(rows1, outb1)
            write(c1, outb1, so1)
            return carry

        lax.fori_loop(0, PAIRS_PER_W, step, 0)
        last0 = base + CHUNKS_PER_W - 2
        wait_write(last0, outb0, so0)
        wait_write(last0 + 1, outb1, so1)

    return gather_kernel


_gather = _make_gather()


def kernel(input, weight):
    wp = jnp.pad(weight, ((0, 0), (0, N_DP - N_D)))
    out_t = _gather(input.T, wp)
    return jnp.transpose(out_t, (2, 0, 1))


# speedup vs baseline: 2.2183x; 1.0747x over previous
import functools

import jax
import jax.numpy as jnp
from jax import lax
from jax.experimental import pallas as pl
from jax.experimental.pallas import tpu as pltpu
from jax.experimental.pallas import tpu_sc as plsc

N_V = 1000000
N_D = 64
N_DP = 128
N_DS = 129
BATCH = 16384
HIST = 50
L = 16

_info = plsc.get_sparse_core_info()
NC, NS = _info.num_cores, _info.num_subcores
NW = NC * NS
CB = 256
NCB = BATCH // CB
N_CHUNKS = HIST * NCB
CHUNKS_PER_W = N_CHUNKS // NW
PAIRS_PER_W = CHUNKS_PER_W // 2


def _make_gather():
    mesh = plsc.VectorSubcoreMesh(core_axis_name="c", subcore_axis_name="s")

    @functools.partial(
        pl.kernel,
        mesh=mesh,
        out_type=jax.ShapeDtypeStruct((HIST, N_D, BATCH), jnp.float32),
        compiler_params=pltpu.CompilerParams(needs_layout_passes=False),
        scratch_types=[
            pltpu.VMEM((CB,), jnp.int32),
            pltpu.VMEM((CB,), jnp.int32),
            pltpu.VMEM((CB, N_DP), jnp.float32),
            pltpu.VMEM((CB, N_DP), jnp.float32),
            pltpu.VMEM((N_D, CB), jnp.float32),
            pltpu.VMEM((N_D, CB), jnp.float32),
            pltpu.SemaphoreType.DMA,
            pltpu.SemaphoreType.DMA,
            pltpu.SemaphoreType.DMA,
            pltpu.SemaphoreType.DMA,
        ],
    )
    def gather_kernel(
        idx_hbm, table_hbm, out_hbm,
        idx0, idx1, rows0, rows1, outb0, outb1,
        sg0, sg1, so0, so1,
    ):
        wid = lax.axis_index("s") * NC + lax.axis_index("c")
        base = wid * CHUNKS_PER_W
        row_iotas = [lax.iota(jnp.int32, L) + jg * L for jg in range(CB // L)]

        def chunk_hb(c):
            h = c // NCB
            b0 = (c % NCB) * CB
            return h, b0

        def stage(c, idx_v, rows_v, sem):
            h, b0 = chunk_hb(c)
            pltpu.sync_copy(idx_hbm.at[h, pl.ds(b0, CB)], idx_v)
            pltpu.async_copy(
                table_hbm.at[idx_v], rows_v.at[slice(None), pl.ds(0, N_DP)], sem
            )

        lane = lax.iota(jnp.int32, L)
        perms = {s: jnp.bitwise_xor(lane, s) for s in (1, 2, 4, 8)}
        masks = {s: jnp.bitwise_and(lane, s) == 0 for s in (1, 2, 4, 8)}

        def transpose(rows_v, outb_v):
            def trans_jb(jb, carry):
                j0 = jb * L
                for dg in range(N_D // L):
                    d0 = dg * L
                    regs = [rows_v[j0 + jj, pl.ds(d0, L)] for jj in range(L)]
                    for s in (1, 2, 4, 8):
                        m, perm = masks[s], perms[s]
                        for p in range(L):
                            if p & s:
                                continue
                            q = p | s
                            a, b = regs[p], regs[q]
                            ta = jnp.take_along_axis(a, perm, axis=0)
                            tb = jnp.take_along_axis(b, perm, axis=0)
                            regs[p] = jnp.where(m, a, tb)
                            regs[q] = jnp.where(m, ta, b)
                    for dd in range(L):
                        outb_v[d0 + dd, pl.ds(j0, L)] = regs[dd]
                return carry

            lax.fori_loop(0, CB // L, trans_jb, 0)

        def write(c, outb_v, sem):
            h, b0 = chunk_hb(c)
            pltpu.async_copy(
                outb_v.at[slice(None), pl.ds(0, CB)],
                out_hbm.at[h, slice(None), pl.ds(b0, CB)],
                sem,
            )

        def wait_gather(c, idx_v, rows_v, sem):
            pltpu.make_async_copy(
                table_hbm.at[idx_v], rows_v.at[slice(None), pl.ds(0, N_DP)], sem
            ).wait()

        def wait_write(c, outb_v, sem):
            h, b0 = chunk_hb(c)
            pltpu.make_async_copy(
                outb_v.at[slice(None), pl.ds(0, CB)],
                out_hbm.at[h, slice(None), pl.ds(b0, CB)],
                sem,
            ).wait()

        stage(base, idx0, rows0, sg0)

        def step(g, carry):
            c0 = base + 2 * g
            c1 = c0 + 1
            c2 = c0 + 2
            stage(c1, idx1, rows1, sg1)
            wait_gather(c0, idx0, rows0, sg0)

            @pl.when(g > 0)
            def _():
                wait_write(c0 - 2, outb0, so0)

            transpose(rows0, outb0)
            write(c0, outb0, so0)

            @pl.when(g < PAIRS_PER_W - 1)
            def _():
                stage(c2, idx0, rows0, sg0)

            wait_gather(c1, idx1, rows1, sg1)

            @pl.when(g > 0)
            def _():
                wait_write(c1 - 2, outb1, so1)

            transpose(rows1, outb1)
            write(c1, outb1, so1)
            return carry

        lax.fori_loop(0, PAIRS_PER_W, step, 0)
        last0 = base + CHUNKS_PER_W - 2
        wait_write(last0, outb0, so0)
        wait_write(last0 + 1, outb1, so1)

    return gather_kernel


_gather = _make_gather()


def kernel(input, weight):
    wp = jnp.pad(weight, ((0, 0), (0, N_DP - N_D)))
    out_t = _gather(input.T, wp)
    return jnp.transpose(out_t, (2, 0, 1))
